# baseline (speedup 1.0000x reference)
import functools

import jax
import jax.numpy as jnp
from jax import lax
from jax.experimental import pallas as pl
from jax.experimental.pallas import tpu as pltpu
from jax.experimental.pallas import tpu_sc as plsc

HASH_BITS = 24
NUM_Q = 16384
FEAT = 512

TC_BLOCK = 1024
TC_GRID = NUM_Q // TC_BLOCK

NUM_CORES = 2
NUM_SUBCORES = 16
NUM_WORKERS = NUM_CORES * NUM_SUBCORES
ROWS_PER_WORKER = NUM_Q // NUM_WORKERS
GATHER_CHUNK = 128
NUM_CHUNKS = ROWS_PER_WORKER // GATHER_CHUNK
LANES = 16


def _hash_tc_body(x_ref, rm_ref, widx_ref, bitpos_ref):
    prod = jnp.dot(x_ref[...], rm_ref[...],
                   preferred_element_type=jnp.float32,
                   precision=jax.lax.Precision.HIGHEST)
    col = lax.broadcasted_iota(jnp.int32, prod.shape, 1)
    pow2 = jnp.where(col < HASH_BITS,
                     lax.shift_left(jnp.int32(1), jnp.minimum(col, HASH_BITS - 1)),
                     0)
    masked = jnp.where(prod < 0.0, pow2, 0)
    idx = jnp.sum(masked, axis=1)
    widx_ref[...] = lax.shift_right_logical(idx, 5)
    bitpos_ref[...] = jnp.bitwise_and(idx, 31)


def _hash_indices(x, rm_pad):
    return pl.pallas_call(
        _hash_tc_body,
        grid=(TC_GRID,),
        in_specs=[
            pl.BlockSpec((TC_BLOCK, FEAT), lambda i: (i, 0)),
            pl.BlockSpec((FEAT, 128), lambda i: (0, 0)),
        ],
        out_specs=[
            pl.BlockSpec((TC_BLOCK,), lambda i: (i,)),
            pl.BlockSpec((TC_BLOCK,), lambda i: (i,)),
        ],
        out_shape=[
            jax.ShapeDtypeStruct((NUM_Q,), jnp.int32),
            jax.ShapeDtypeStruct((NUM_Q,), jnp.int32),
        ],
    )(x, rm_pad)


def _lookup_sc_body(widx_hbm, bitpos_hbm, words_hbm, out_hbm,
                    widx_v, bitpos_v, words_v, out_v, sem):
    wid = lax.axis_index("s") * NUM_CORES + lax.axis_index("c")
    base = wid * ROWS_PER_WORKER
    pltpu.sync_copy(widx_hbm.at[pl.ds(base, ROWS_PER_WORKER)], widx_v)
    pltpu.sync_copy(bitpos_hbm.at[pl.ds(base, ROWS_PER_WORKER)], bitpos_v)
    copies = []
    for j in range(NUM_CHUNKS):
        sl = pl.ds(j * GATHER_CHUNK, GATHER_CHUNK)
        copies.append(pltpu.async_copy(words_hbm.at[widx_v.at[sl]],
                                       words_v.at[sl], sem))
    for c in copies:
        c.wait()
    for i in range(ROWS_PER_WORKER // LANES):
        sl = pl.ds(i * LANES, LANES)
        out_v[sl] = jnp.bitwise_and(
            lax.shift_right_logical(words_v[sl], bitpos_v[sl]), 1)
    pltpu.sync_copy(out_v, out_hbm.at[pl.ds(base, ROWS_PER_WORKER)])


@functools.cache
def _lookup_bits_kernel():
    return pl.kernel(
        _lookup_sc_body,
        out_type=jax.ShapeDtypeStruct((NUM_Q,), jnp.int32),
        mesh=plsc.VectorSubcoreMesh(core_axis_name="c", subcore_axis_name="s",
                                    num_cores=NUM_CORES,
                                    num_subcores=NUM_SUBCORES),
        scratch_types=[
            pltpu.VMEM((ROWS_PER_WORKER,), jnp.int32),
            pltpu.VMEM((ROWS_PER_WORKER,), jnp.int32),
            pltpu.VMEM((ROWS_PER_WORKER,), jnp.int32),
            pltpu.VMEM((ROWS_PER_WORKER,), jnp.int32),
            pltpu.SemaphoreType.DMA,
        ],
    )


def kernel(x, is_training, test_local_stats, random_matrix, binary_set):
    x = jnp.reshape(x, (x.shape[0], -1))
    rm = jax.lax.stop_gradient(random_matrix)
    rm_pad = jnp.pad(rm, ((0, 0), (0, 128 - HASH_BITS)))
    words = lax.bitcast_convert_type(
        jnp.reshape(binary_set, (-1, 4)), jnp.int32)
    widx, bitpos = _hash_indices(x, rm_pad)
    bits = _lookup_bits_kernel()(widx, bitpos, words)
    return bits.astype(jnp.bool_)


# speedup vs baseline: 10.1240x; 10.1240x over previous
import functools

import jax
import jax.numpy as jnp
from jax import lax
from jax.experimental import pallas as pl
from jax.experimental.pallas import tpu as pltpu
from jax.experimental.pallas import tpu_sc as plsc

HASH_BITS = 24
NUM_Q = 16384
FEAT = 512
NUM_BYTES = 2 ** (HASH_BITS - 3)
NUM_WORDS = 2 ** (HASH_BITS - 5)

TC_BLOCK = 1024
TC_GRID = NUM_Q // TC_BLOCK
BYTE_ROWS = NUM_BYTES // 128
BYTE_ROWS_BLK = BYTE_ROWS // TC_GRID
WORDS_BLK = NUM_WORDS // TC_GRID

NUM_CORES = 2
NUM_SUBCORES = 16
NUM_WORKERS = NUM_CORES * NUM_SUBCORES
ROWS_PER_WORKER = NUM_Q // NUM_WORKERS
GATHER_CHUNK = 128
NUM_CHUNKS = ROWS_PER_WORKER // GATHER_CHUNK
LANES = 16


def _hash_tc_body(x_ref, rm_ref, pow2_ref, bset_ref,
                  widx_ref, bitpos_ref, words_ref):
    words_ref[...] = jnp.reshape(
        pltpu.bitcast(bset_ref[...], jnp.int32), (WORDS_BLK,))

    prod = jnp.dot(x_ref[...], rm_ref[...],
                   preferred_element_type=jnp.float32)
    signs = (prod < 0.0).astype(jnp.bfloat16)
    idx_f = lax.dot_general(pow2_ref[...], signs,
                            (((1,), (1,)), ((), ())),
                            preferred_element_type=jnp.float32)
    h = idx_f.astype(jnp.int32)
    b = lax.shift_right_logical(h, 3)
    widx = jnp.bitwise_or(
        lax.shift_left(lax.shift_right_logical(b, 9), 7),
        jnp.bitwise_and(b, 127))
    k = jnp.bitwise_and(lax.shift_right_logical(b, 7), 3)
    bitpos = jnp.bitwise_or(lax.shift_left(k, 3), jnp.bitwise_and(h, 7))
    widx_ref[...] = jnp.reshape(widx, (1, 1, TC_BLOCK))
    bitpos_ref[...] = jnp.reshape(bitpos, (1, 1, TC_BLOCK))


def _hash_and_repack(x, rm_pad, pow2, bset2d):
    return pl.pallas_call(
        _hash_tc_body,
        grid=(TC_GRID,),
        in_specs=[
            pl.BlockSpec((TC_BLOCK, FEAT), lambda i: (i, 0)),
            pl.BlockSpec((FEAT, 128), lambda i: (0, 0)),
            pl.BlockSpec((1, 128), lambda i: (0, 0)),
            pl.BlockSpec((BYTE_ROWS_BLK, 128), lambda i: (i, 0)),
        ],
        out_specs=[
            pl.BlockSpec((1, 1, TC_BLOCK), lambda i: (i, 0, 0)),
            pl.BlockSpec((1, 1, TC_BLOCK), lambda i: (i, 0, 0)),
            pl.BlockSpec((WORDS_BLK,), lambda i: (i,)),
        ],
        out_shape=[
            jax.ShapeDtypeStruct((TC_GRID, 1, TC_BLOCK), jnp.int32),
            jax.ShapeDtypeStruct((TC_GRID, 1, TC_BLOCK), jnp.int32),
            jax.ShapeDtypeStruct((NUM_WORDS,), jnp.int32),
        ],
    )(x, rm_pad, pow2, bset2d)


def _lookup_sc_body(widx_hbm, bitpos_hbm, words_hbm, out_hbm,
                    widx_v, bitpos_v, words_v, out_v, sem):
    wid = lax.axis_index("s") * NUM_CORES + lax.axis_index("c")
    base = wid * ROWS_PER_WORKER
    row = base // TC_BLOCK
    col = base % TC_BLOCK
    pltpu.sync_copy(widx_hbm.at[row, 0, pl.ds(col, ROWS_PER_WORKER)], widx_v)
    pltpu.sync_copy(bitpos_hbm.at[row, 0, pl.ds(col, ROWS_PER_WORKER)], bitpos_v)
    copies = []
    for j in range(NUM_CHUNKS):
        sl = pl.ds(j * GATHER_CHUNK, GATHER_CHUNK)
        copies.append(pltpu.async_copy(words_hbm.at[widx_v.at[sl]],
                                       words_v.at[sl], sem))
    for c in copies:
        c.wait()
    for i in range(ROWS_PER_WORKER // LANES):
        sl = pl.ds(i * LANES, LANES)
        out_v[sl] = jnp.bitwise_and(
            lax.shift_right_logical(words_v[sl], bitpos_v[sl]), 1)
    pltpu.sync_copy(out_v, out_hbm.at[pl.ds(base, ROWS_PER_WORKER)])


@functools.cache
def _lookup_bits_kernel():
    return pl.kernel(
        _lookup_sc_body,
        out_type=jax.ShapeDtypeStruct((NUM_Q,), jnp.int32),
        mesh=plsc.VectorSubcoreMesh(core_axis_name="c", subcore_axis_name="s",
                                    num_cores=NUM_CORES,
                                    num_subcores=NUM_SUBCORES),
        scratch_types=[
            pltpu.VMEM((ROWS_PER_WORKER,), jnp.int32),
            pltpu.VMEM((ROWS_PER_WORKER,), jnp.int32),
            pltpu.VMEM((ROWS_PER_WORKER,), jnp.int32),
            pltpu.VMEM((ROWS_PER_WORKER,), jnp.int32),
            pltpu.SemaphoreType.DMA,
        ],
    )


def kernel(x, is_training, test_local_stats, random_matrix, binary_set):
    x = jnp.reshape(x, (x.shape[0], -1))
    rm = jax.lax.stop_gradient(random_matrix)
    rm_pad = jnp.pad(rm, ((0, 0), (0, 128 - HASH_BITS)))
    col = jnp.arange(128, dtype=jnp.int32)[None, :]
    pow2 = jnp.where(col < HASH_BITS,
                     jnp.exp2(col.astype(jnp.float32)), 0.0
                     ).astype(jnp.bfloat16)
    bset2d = jnp.reshape(binary_set, (BYTE_ROWS, 128))
    widx, bitpos, words = _hash_and_repack(x, rm_pad, pow2, bset2d)
    bits = _lookup_bits_kernel()(widx, bitpos, words)
    return bits.astype(jnp.bool_)


# speedup vs baseline: 10.3564x; 1.0230x over previous
import functools

import jax
import jax.numpy as jnp
from jax import lax
from jax.experimental import pallas as pl
from jax.experimental.pallas import tpu as pltpu
from jax.experimental.pallas import tpu_sc as plsc

HASH_BITS = 24
NUM_Q = 16384
FEAT = 512
NUM_BYTES = 2 ** (HASH_BITS - 3)
NUM_WORDS = 2 ** (HASH_BITS - 5)

TC_BLOCK = 1024
TC_GRID = NUM_Q // TC_BLOCK
BYTE_ROWS = NUM_BYTES // 128
BYTE_ROWS_BLK = BYTE_ROWS // TC_GRID
WORDS_BLK = NUM_WORDS // TC_GRID

NUM_CORES = 2
NUM_SUBCORES = 16
NUM_WORKERS = NUM_CORES * NUM_SUBCORES
ROWS_PER_WORKER = NUM_Q // NUM_WORKERS
GATHER_CHUNK = 128
NUM_CHUNKS = ROWS_PER_WORKER // GATHER_CHUNK
LANES = 16


def _hash_tc_body(x_ref, rm_ref, bset_ref, packed_ref, words_ref):
    words_ref[...] = jnp.reshape(
        pltpu.bitcast(bset_ref[...], jnp.int32), (WORDS_BLK,))

    prod = jnp.dot(x_ref[...], rm_ref[...],
                   preferred_element_type=jnp.float32)
    signs = (prod < 0.0).astype(jnp.bfloat16)
    col = lax.broadcasted_iota(jnp.int32, (1, HASH_BITS), 1)
    pow2 = lax.shift_left(jnp.int32(1), col).astype(jnp.bfloat16)
    idx_f = lax.dot_general(pow2, signs,
                            (((1,), (1,)), ((), ())),
                            preferred_element_type=jnp.float32)
    h = idx_f.astype(jnp.int32)
    b = lax.shift_right_logical(h, 3)
    widx = jnp.bitwise_or(
        lax.shift_left(lax.shift_right_logical(b, 9), 7),
        jnp.bitwise_and(b, 127))
    k = jnp.bitwise_and(lax.shift_right_logical(b, 7), 3)
    bitpos = jnp.bitwise_or(lax.shift_left(k, 3), jnp.bitwise_and(h, 7))
    packed = jnp.bitwise_or(lax.shift_left(widx, 5), bitpos)
    packed_ref[...] = jnp.reshape(packed, (1, 1, TC_BLOCK))


def _hash_and_repack(x, rm, bset2d):
    return pl.pallas_call(
        _hash_tc_body,
        grid=(TC_GRID,),
        in_specs=[
            pl.BlockSpec((TC_BLOCK, FEAT), lambda i: (i, 0)),
            pl.BlockSpec((FEAT, HASH_BITS), lambda i: (0, 0)),
            pl.BlockSpec((BYTE_ROWS_BLK, 128), lambda i: (i, 0)),
        ],
        out_specs=[
            pl.BlockSpec((1, 1, TC_BLOCK), lambda i: (i, 0, 0)),
            pl.BlockSpec((WORDS_BLK,), lambda i: (i,)),
        ],
        out_shape=[
            jax.ShapeDtypeStruct((TC_GRID, 1, TC_BLOCK), jnp.int32),
            jax.ShapeDtypeStruct((NUM_WORDS,), jnp.int32),
        ],
    )(x, rm, bset2d)


def _lookup_sc_body(packed_hbm, words_hbm, out_hbm,
                    packed_v, widx_v, words_v, out_v, sem):
    wid = lax.axis_index("s") * NUM_CORES + lax.axis_index("c")
    base = wid * ROWS_PER_WORKER
    row = base // TC_BLOCK
    col = base % TC_BLOCK
    pltpu.sync_copy(packed_hbm.at[row, 0, pl.ds(col, ROWS_PER_WORKER)],
                    packed_v)
    for i in range(ROWS_PER_WORKER // LANES):
        sl = pl.ds(i * LANES, LANES)
        widx_v[sl] = lax.shift_right_logical(packed_v[sl], 5)
    copies = []
    for j in range(NUM_CHUNKS):
        sl = pl.ds(j * GATHER_CHUNK, GATHER_CHUNK)
        copies.append(pltpu.async_copy(words_hbm.at[widx_v.at[sl]],
                                       words_v.at[sl], sem))
    for c in copies:
        c.wait()
    for i in range(ROWS_PER_WORKER // LANES):
        sl = pl.ds(i * LANES, LANES)
        out_v[sl] = jnp.bitwise_and(
            lax.shift_right_logical(words_v[sl],
                                    jnp.bitwise_and(packed_v[sl], 31)), 1)
    pltpu.sync_copy(out_v, out_hbm.at[pl.ds(base, ROWS_PER_WORKER)])


@functools.cache
def _lookup_bits_kernel():
    return pl.kernel(
        _lookup_sc_body,
        out_type=jax.ShapeDtypeStruct((NUM_Q,), jnp.int32),
        mesh=plsc.VectorSubcoreMesh(core_axis_name="c", subcore_axis_name="s",
                                    num_cores=NUM_CORES,
                                    num_subcores=NUM_SUBCORES),
        scratch_types=[
            pltpu.VMEM((ROWS_PER_WORKER,), jnp.int32),
            pltpu.VMEM((ROWS_PER_WORKER,), jnp.int32),
            pltpu.VMEM((ROWS_PER_WORKER,), jnp.int32),
            pltpu.VMEM((ROWS_PER_WORKER,), jnp.int32),
            pltpu.SemaphoreType.DMA,
        ],
    )


def kernel(x, is_training, test_local_stats, random_matrix, binary_set):
    x = jnp.reshape(x, (x.shape[0], -1))
    rm = jax.lax.stop_gradient(random_matrix)
    bset2d = jnp.reshape(binary_set, (BYTE_ROWS, 128))
    packed, words = _hash_and_repack(x, rm, bset2d)
    bits = _lookup_bits_kernel()(packed, words)
    return bits.astype(jnp.bool_)


# speedup vs baseline: 11.5120x; 1.1116x over previous
import functools

import jax
import jax.numpy as jnp
from jax import lax
from jax.experimental import pallas as pl
from jax.experimental.pallas import tpu as pltpu
from jax.experimental.pallas import tpu_sc as plsc

HASH_BITS = 24
NUM_Q = 16384
FEAT = 512
NUM_BYTES = 2 ** (HASH_BITS - 3)
NUM_WORDS = 2 ** (HASH_BITS - 5)

TC_BLOCK = 2048
TC_GRID = NUM_Q // TC_BLOCK
BYTE_ROWS = NUM_BYTES // 128
BYTE_ROWS_BLK = BYTE_ROWS // TC_GRID
WORDS_BLK = NUM_WORDS // TC_GRID

NUM_CORES = 2
NUM_SUBCORES = 16
NUM_WORKERS = NUM_CORES * NUM_SUBCORES
ROWS_PER_WORKER = NUM_Q // NUM_WORKERS
GATHER_CHUNK = 128
NUM_CHUNKS = ROWS_PER_WORKER // GATHER_CHUNK
LANES = 16


def _hash_tc_body(x_ref, rm_ref, bset_ref, packed_ref, words_ref):
    words_ref[...] = jnp.reshape(
        pltpu.bitcast(bset_ref[...], jnp.int32), (WORDS_BLK,))

    prod = jnp.dot(x_ref[...], rm_ref[...],
                   preferred_element_type=jnp.float32)
    signs = (prod < 0.0).astype(jnp.bfloat16)
    col = lax.broadcasted_iota(jnp.int32, (1, HASH_BITS), 1)
    pow2 = lax.shift_left(jnp.int32(1), col).astype(jnp.bfloat16)
    idx_f = lax.dot_general(pow2, signs,
                            (((1,), (1,)), ((), ())),
                            preferred_element_type=jnp.float32)
    h = idx_f.astype(jnp.int32)
    b = lax.shift_right_logical(h, 3)
    widx = jnp.bitwise_or(
        lax.shift_left(lax.shift_right_logical(b, 9), 7),
        jnp.bitwise_and(b, 127))
    k = jnp.bitwise_and(lax.shift_right_logical(b, 7), 3)
    bitpos = jnp.bitwise_or(lax.shift_left(k, 3), jnp.bitwise_and(h, 7))
    packed = jnp.bitwise_or(lax.shift_left(widx, 5), bitpos)
    packed_ref[...] = jnp.reshape(packed, (1, 1, TC_BLOCK))


def _hash_and_repack(x, rm, bset2d):
    return pl.pallas_call(
        _hash_tc_body,
        grid=(TC_GRID,),
        in_specs=[
            pl.BlockSpec((TC_BLOCK, FEAT), lambda i: (i, 0)),
            pl.BlockSpec((FEAT, HASH_BITS), lambda i: (0, 0)),
            pl.BlockSpec((BYTE_ROWS_BLK, 128), lambda i: (i, 0)),
        ],
        out_specs=[
            pl.BlockSpec((1, 1, TC_BLOCK), lambda i: (i, 0, 0)),
            pl.BlockSpec((WORDS_BLK,), lambda i: (i,)),
        ],
        out_shape=[
            jax.ShapeDtypeStruct((TC_GRID, 1, TC_BLOCK), jnp.int32),
            jax.ShapeDtypeStruct((NUM_WORDS,), jnp.int32),
        ],
    )(x, rm, bset2d)


def _lookup_sc_body(packed_hbm, words_hbm, out_hbm,
                    packed_v, widx_v, words_v, out_v, sem):
    wid = lax.axis_index("s") * NUM_CORES + lax.axis_index("c")
    base = wid * ROWS_PER_WORKER
    row = base // TC_BLOCK
    col = base % TC_BLOCK
    pltpu.sync_copy(packed_hbm.at[row, 0, pl.ds(col, ROWS_PER_WORKER)],
                    packed_v)
    for i in range(ROWS_PER_WORKER // LANES):
        sl = pl.ds(i * LANES, LANES)
        widx_v[sl] = lax.shift_right_logical(packed_v[sl], 5)
    copies = []
    for j in range(NUM_CHUNKS):
        sl = pl.ds(j * GATHER_CHUNK, GATHER_CHUNK)
        copies.append(pltpu.async_copy(words_hbm.at[widx_v.at[sl]],
                                       words_v.at[sl], sem))
    for c in copies:
        c.wait()
    for i in range(ROWS_PER_WORKER // LANES):
        sl = pl.ds(i * LANES, LANES)
        out_v[sl] = jnp.bitwise_and(
            lax.shift_right_logical(words_v[sl],
                                    jnp.bitwise_and(packed_v[sl], 31)), 1)
    pltpu.sync_copy(out_v, out_hbm.at[pl.ds(base, ROWS_PER_WORKER)])


@functools.cache
def _lookup_bits_kernel():
    return pl.kernel(
        _lookup_sc_body,
        out_type=jax.ShapeDtypeStruct((NUM_Q,), jnp.int32),
        mesh=plsc.VectorSubcoreMesh(core_axis_name="c", subcore_axis_name="s",
                                    num_cores=NUM_CORES,
                                    num_subcores=NUM_SUBCORES),
        scratch_types=[
            pltpu.VMEM((ROWS_PER_WORKER,), jnp.int32),
            pltpu.VMEM((ROWS_PER_WORKER,), jnp.int32),
            pltpu.VMEM((ROWS_PER_WORKER,), jnp.int32),
            pltpu.VMEM((ROWS_PER_WORKER,), jnp.int32),
            pltpu.SemaphoreType.DMA,
        ],
    )


def kernel(x, is_training, test_local_stats, random_matrix, binary_set):
    x = jnp.reshape(x, (x.shape[0], -1))
    rm = jax.lax.stop_gradient(random_matrix)
    bset2d = jnp.reshape(binary_set, (BYTE_ROWS, 128))
    packed, words = _hash_and_repack(x, rm, bset2d)
    bits = _lookup_bits_kernel()(packed, words)
    return bits.astype(jnp.bool_)


# speedup vs baseline: 11.5874x; 1.0065x over previous
import functools

import jax
import jax.numpy as jnp
from jax import lax
from jax.experimental import pallas as pl
from jax.experimental.pallas import tpu as pltpu
from jax.experimental.pallas import tpu_sc as plsc

HASH_BITS = 24
NUM_Q = 16384
FEAT = 512
NUM_BYTES = 2 ** (HASH_BITS - 3)
NUM_WORDS = 2 ** (HASH_BITS - 5)

TC_BLOCK = 4096
TC_GRID = NUM_Q // TC_BLOCK
BYTE_ROWS = NUM_BYTES // 128
BYTE_ROWS_BLK = BYTE_ROWS // TC_GRID
WORDS_BLK = NUM_WORDS // TC_GRID

NUM_CORES = 2
NUM_SUBCORES = 16
NUM_WORKERS = NUM_CORES * NUM_SUBCORES
ROWS_PER_WORKER = NUM_Q // NUM_WORKERS
GATHER_CHUNK = 128
NUM_CHUNKS = ROWS_PER_WORKER // GATHER_CHUNK
LANES = 16


def _hash_tc_body(x_ref, rm_ref, bset_ref, packed_ref, words_ref):
    words_ref[...] = jnp.reshape(
        pltpu.bitcast(bset_ref[...], jnp.int32), (WORDS_BLK,))

    prod = jnp.dot(x_ref[...], rm_ref[...],
                   preferred_element_type=jnp.float32)
    signs = (prod < 0.0).astype(jnp.bfloat16)
    col = lax.broadcasted_iota(jnp.int32, (1, HASH_BITS), 1)
    pow2 = lax.shift_left(jnp.int32(1), col).astype(jnp.bfloat16)
    idx_f = lax.dot_general(pow2, signs,
                            (((1,), (1,)), ((), ())),
                            preferred_element_type=jnp.float32)
    h = idx_f.astype(jnp.int32)
    b = lax.shift_right_logical(h, 3)
    widx = jnp.bitwise_or(
        lax.shift_left(lax.shift_right_logical(b, 9), 7),
        jnp.bitwise_and(b, 127))
    k = jnp.bitwise_and(lax.shift_right_logical(b, 7), 3)
    bitpos = jnp.bitwise_or(lax.shift_left(k, 3), jnp.bitwise_and(h, 7))
    packed = jnp.bitwise_or(lax.shift_left(widx, 5), bitpos)
    packed_ref[...] = jnp.reshape(packed, (1, 1, TC_BLOCK))


def _hash_and_repack(x, rm, bset2d):
    return pl.pallas_call(
        _hash_tc_body,
        grid=(TC_GRID,),
        in_specs=[
            pl.BlockSpec((TC_BLOCK, FEAT), lambda i: (i, 0)),
            pl.BlockSpec((FEAT, HASH_BITS), lambda i: (0, 0)),
            pl.BlockSpec((BYTE_ROWS_BLK, 128), lambda i: (i, 0)),
        ],
        out_specs=[
            pl.BlockSpec((1, 1, TC_BLOCK), lambda i: (i, 0, 0)),
            pl.BlockSpec((WORDS_BLK,), lambda i: (i,)),
        ],
        out_shape=[
            jax.ShapeDtypeStruct((TC_GRID, 1, TC_BLOCK), jnp.int32),
            jax.ShapeDtypeStruct((NUM_WORDS,), jnp.int32),
        ],
    )(x, rm, bset2d)


def _lookup_sc_body(packed_hbm, words_hbm, out_hbm,
                    packed_v, widx_v, words_v, out_v, sem):
    wid = lax.axis_index("s") * NUM_CORES + lax.axis_index("c")
    base = wid * ROWS_PER_WORKER
    row = base // TC_BLOCK
    col = base % TC_BLOCK
    pltpu.sync_copy(packed_hbm.at[row, 0, pl.ds(col, ROWS_PER_WORKER)],
                    packed_v)
    for i in range(ROWS_PER_WORKER // LANES):
        sl = pl.ds(i * LANES, LANES)
        widx_v[sl] = lax.shift_right_logical(packed_v[sl], 5)
    copies = []
    for j in range(NUM_CHUNKS):
        sl = pl.ds(j * GATHER_CHUNK, GATHER_CHUNK)
        copies.append(pltpu.async_copy(words_hbm.at[widx_v.at[sl]],
                                       words_v.at[sl], sem))
    for c in copies:
        c.wait()
    for i in range(ROWS_PER_WORKER // LANES):
        sl = pl.ds(i * LANES, LANES)
        out_v[sl] = jnp.bitwise_and(
            lax.shift_right_logical(words_v[sl],
                                    jnp.bitwise_and(packed_v[sl], 31)), 1)
    pltpu.sync_copy(out_v, out_hbm.at[pl.ds(base, ROWS_PER_WORKER)])


@functools.cache
def _lookup_bits_kernel():
    return pl.kernel(
        _lookup_sc_body,
        out_type=jax.ShapeDtypeStruct((NUM_Q,), jnp.int32),
        mesh=plsc.VectorSubcoreMesh(core_axis_name="c", subcore_axis_name="s",
                                    num_cores=NUM_CORES,
                                    num_subcores=NUM_SUBCORES),
        scratch_types=[
            pltpu.VMEM((ROWS_PER_WORKER,), jnp.int32),
            pltpu.VMEM((ROWS_PER_WORKER,), jnp.int32),
            pltpu.VMEM((ROWS_PER_WORKER,), jnp.int32),
            pltpu.VMEM((ROWS_PER_WORKER,), jnp.int32),
            pltpu.SemaphoreType.DMA,
        ],
    )


def kernel(x, is_training, test_local_stats, random_matrix, binary_set):
    x = jnp.reshape(x, (x.shape[0], -1))
    rm = jax.lax.stop_gradient(random_matrix)
    bset2d = jnp.reshape(binary_set, (BYTE_ROWS, 128))
    packed, words = _hash_and_repack(x, rm, bset2d)
    bits = _lookup_bits_kernel()(packed, words)
    return bits.astype(jnp.bool_)


# speedup vs baseline: 11.6466x; 1.0051x over previous
import functools

import jax
import jax.numpy as jnp
from jax import lax
from jax.experimental import pallas as pl
from jax.experimental.pallas import tpu as pltpu
from jax.experimental.pallas import tpu_sc as plsc

HASH_BITS = 24
NUM_Q = 16384
FEAT = 512
NUM_BYTES = 2 ** (HASH_BITS - 3)
NUM_WORDS = 2 ** (HASH_BITS - 5)

TC_BLOCK = 4096
TC_GRID = NUM_Q // TC_BLOCK
BYTE_ROWS = NUM_BYTES // 128
BYTE_ROWS_BLK = BYTE_ROWS // TC_GRID
WORDS_BLK = NUM_WORDS // TC_GRID

NUM_CORES = 2
NUM_SUBCORES = 16
NUM_WORKERS = NUM_CORES * NUM_SUBCORES
ROWS_PER_WORKER = NUM_Q // NUM_WORKERS
GATHER_CHUNK = 128
NUM_CHUNKS = ROWS_PER_WORKER // GATHER_CHUNK
LANES = 16


def _hash_tc_body(x_ref, rm_ref, bset_ref, packed_ref, words_ref):
    bblk = jnp.reshape(bset_ref[...], (BYTE_ROWS_BLK, 128))
    words_ref[...] = jnp.reshape(pltpu.bitcast(bblk, jnp.int32), (WORDS_BLK,))

    prod = jnp.dot(x_ref[...], rm_ref[...],
                   preferred_element_type=jnp.float32)
    signs = (prod < 0.0).astype(jnp.bfloat16)
    col = lax.broadcasted_iota(jnp.int32, (1, HASH_BITS), 1)
    pow2 = lax.shift_left(jnp.int32(1), col).astype(jnp.bfloat16)
    idx_f = lax.dot_general(pow2, signs,
                            (((1,), (1,)), ((), ())),
                            preferred_element_type=jnp.float32)
    h = idx_f.astype(jnp.int32)
    b = lax.shift_right_logical(h, 3)
    widx = jnp.bitwise_or(
        lax.shift_left(lax.shift_right_logical(b, 9), 7),
        jnp.bitwise_and(b, 127))
    k = jnp.bitwise_and(lax.shift_right_logical(b, 7), 3)
    bitpos = jnp.bitwise_or(lax.shift_left(k, 3), jnp.bitwise_and(h, 7))
    packed = jnp.bitwise_or(lax.shift_left(widx, 5), bitpos)
    packed_ref[...] = jnp.reshape(packed, (1, 1, TC_BLOCK))


def _hash_and_repack(x, rm, bset2d):
    return pl.pallas_call(
        _hash_tc_body,
        grid=(TC_GRID,),
        in_specs=[
            pl.BlockSpec((TC_BLOCK, FEAT), lambda i: (i, 0)),
            pl.BlockSpec((FEAT, HASH_BITS), lambda i: (0, 0)),
            pl.BlockSpec((NUM_BYTES // TC_GRID,), lambda i: (i,)),
        ],
        out_specs=[
            pl.BlockSpec((1, 1, TC_BLOCK), lambda i: (i, 0, 0)),
            pl.BlockSpec((WORDS_BLK,), lambda i: (i,)),
        ],
        out_shape=[
            jax.ShapeDtypeStruct((TC_GRID, 1, TC_BLOCK), jnp.int32),
            jax.ShapeDtypeStruct((NUM_WORDS,), jnp.int32),
        ],
    )(x, rm, bset2d)


def _lookup_sc_body(packed_hbm, words_hbm, out_hbm,
                    packed_v, widx_v, words_v, out_v, sem):
    wid = lax.axis_index("s") * NUM_CORES + lax.axis_index("c")
    base = wid * ROWS_PER_WORKER
    row = base // TC_BLOCK
    col = base % TC_BLOCK
    pltpu.sync_copy(packed_hbm.at[row, 0, pl.ds(col, ROWS_PER_WORKER)],
                    packed_v)
    for i in range(ROWS_PER_WORKER // LANES):
        sl = pl.ds(i * LANES, LANES)
        widx_v[sl] = lax.shift_right_logical(packed_v[sl], 5)
    copies = []
    for j in range(NUM_CHUNKS):
        sl = pl.ds(j * GATHER_CHUNK, GATHER_CHUNK)
        copies.append(pltpu.async_copy(words_hbm.at[widx_v.at[sl]],
                                       words_v.at[sl], sem))
    for c in copies:
        c.wait()
    for i in range(ROWS_PER_WORKER // LANES):
        sl = pl.ds(i * LANES, LANES)
        out_v[sl] = jnp.bitwise_and(
            lax.shift_right_logical(words_v[sl],
                                    jnp.bitwise_and(packed_v[sl], 31)), 1)
    pltpu.sync_copy(out_v, out_hbm.at[pl.ds(base, ROWS_PER_WORKER)])


@functools.cache
def _lookup_bits_kernel():
    return pl.kernel(
        _lookup_sc_body,
        out_type=jax.ShapeDtypeStruct((NUM_Q,), jnp.int32),
        mesh=plsc.VectorSubcoreMesh(core_axis_name="c", subcore_axis_name="s",
                                    num_cores=NUM_CORES,
                                    num_subcores=NUM_SUBCORES),
        scratch_types=[
            pltpu.VMEM((ROWS_PER_WORKER,), jnp.int32),
            pltpu.VMEM((ROWS_PER_WORKER,), jnp.int32),
            pltpu.VMEM((ROWS_PER_WORKER,), jnp.int32),
            pltpu.VMEM((ROWS_PER_WORKER,), jnp.int32),
            pltpu.SemaphoreType.DMA,
        ],
    )


def kernel(x, is_training, test_local_stats, random_matrix, binary_set):
    x = jnp.reshape(x, (x.shape[0], -1))
    rm = jax.lax.stop_gradient(random_matrix)
    packed, words = _hash_and_repack(x, rm, binary_set)
    bits = _lookup_bits_kernel()(packed, words)
    return bits.astype(jnp.bool_)
